# Initial kernel scaffold; baseline (speedup 1.0000x reference)
import jax
import jax.numpy as jnp
from jax.experimental import pallas as pl


def kernel(class_indices, templates):
    raise NotImplementedError("write your pallas kernel here")


# speedup vs baseline: 4.7320x; 4.7320x over previous
import functools

import jax
import jax.numpy as jnp
from jax import lax
from jax.experimental import pallas as pl
from jax.experimental.pallas import tpu as pltpu
from jax.experimental.pallas import tpu_sc as plsc

_NC = 2
_NS = 16
_NW = _NC * _NS
_L = 16


def _lookup_body(idx_hbm, tab_hbm, out_hbm, idx_v, out_v, tab_v,
                 *, n_per_w, chunk, row):
    wid = lax.axis_index("s") * _NC + lax.axis_index("c")
    pltpu.sync_copy(tab_hbm, tab_v)

    base = wid * n_per_w
    num_chunks = n_per_w // chunk
    groups = chunk // _L
    siota = lax.iota(jnp.int32, _L) * row

    def do_chunk(ci, carry):
        cbase = base + ci * chunk
        pltpu.sync_copy(idx_hbm.at[pl.ds(cbase, chunk)], idx_v)

        def do_group(g, c2):
            a0 = idx_v[pl.ds(g * _L, _L)] * row
            sbase = siota + g * (_L * row)
            for j in range(row):
                vals = plsc.load_gather(tab_v, [a0 + j])
                plsc.store_scatter(out_v, [sbase + j], vals)
            return c2

        lax.fori_loop(0, groups, do_group, 0)
        pltpu.sync_copy(out_v, out_hbm.at[pl.ds(cbase * row, chunk * row)])
        return carry

    lax.fori_loop(0, num_chunks, do_chunk, 0)


def kernel(class_indices, templates):
    B, C, H, W = class_indices.shape
    V, t0, t1 = templates.shape
    row = t0 * t1
    N = B * C * H * W
    assert N % _NW == 0
    n_per_w = N // _NW

    chunk = 4096
    while n_per_w % chunk:
        chunk //= 2

    flat_idx = class_indices.reshape(N).astype(jnp.int32)
    tab = templates.reshape(V * row)

    mesh = plsc.VectorSubcoreMesh(
        core_axis_name="c", subcore_axis_name="s",
        num_cores=_NC, num_subcores=_NS)

    out = pl.kernel(
        functools.partial(_lookup_body, n_per_w=n_per_w, chunk=chunk,
                          row=row),
        out_type=jax.ShapeDtypeStruct((N * row,), jnp.float32),
        mesh=mesh,
        compiler_params=pltpu.CompilerParams(needs_layout_passes=False),
        scratch_types=[
            pltpu.VMEM((chunk,), jnp.int32),
            pltpu.VMEM((chunk * row,), jnp.float32),
            pltpu.VMEM((V * row,), jnp.float32),
        ],
    )(flat_idx, tab)

    return out.reshape(B, C, H, W, t0, t1)


# speedup vs baseline: 4.9236x; 1.0405x over previous
import functools

import jax
import jax.numpy as jnp
from jax import lax
from jax.experimental import pallas as pl
from jax.experimental.pallas import tpu as pltpu
from jax.experimental.pallas import tpu_sc as plsc

_NC = 2
_NS = 16
_NW = _NC * _NS
_L = 16


def _lookup_body(idx_hbm, tab_hbm, out_hbm, idx_v, out_v, tab_v,
                 *, n_per_w, chunk, row):
    wid = lax.axis_index("s") * _NC + lax.axis_index("c")
    pltpu.sync_copy(tab_hbm, tab_v)

    base = wid * n_per_w
    num_chunks = n_per_w // chunk
    groups = chunk // _L
    siota = lax.iota(jnp.int32, _L) * row

    def do_chunk(ci, carry):
        cbase = base + ci * chunk
        pltpu.sync_copy(idx_hbm.at[pl.ds(cbase, chunk)], idx_v)

        @plsc.parallel_loop(0, groups, unroll=4)
        def do_group(g):
            a0 = idx_v[pl.ds(g * _L, _L)] * row
            sbase = siota + g * (_L * row)
            for j in range(row):
                vals = plsc.load_gather(tab_v, [a0 + j])
                plsc.store_scatter(out_v, [sbase + j], vals)
        pltpu.sync_copy(out_v, out_hbm.at[pl.ds(cbase * row, chunk * row)])
        return carry

    lax.fori_loop(0, num_chunks, do_chunk, 0)


def kernel(class_indices, templates):
    B, C, H, W = class_indices.shape
    V, t0, t1 = templates.shape
    row = t0 * t1
    N = B * C * H * W
    assert N % _NW == 0
    n_per_w = N // _NW

    chunk = 4096
    while n_per_w % chunk:
        chunk //= 2

    flat_idx = class_indices.reshape(N).astype(jnp.int32)
    tab = templates.reshape(V * row)

    mesh = plsc.VectorSubcoreMesh(
        core_axis_name="c", subcore_axis_name="s",
        num_cores=_NC, num_subcores=_NS)

    out = pl.kernel(
        functools.partial(_lookup_body, n_per_w=n_per_w, chunk=chunk,
                          row=row),
        out_type=jax.ShapeDtypeStruct((N * row,), jnp.float32),
        mesh=mesh,
        compiler_params=pltpu.CompilerParams(needs_layout_passes=False),
        scratch_types=[
            pltpu.VMEM((chunk,), jnp.int32),
            pltpu.VMEM((chunk * row,), jnp.float32),
            pltpu.VMEM((V * row,), jnp.float32),
        ],
    )(flat_idx, tab)

    return out.reshape(B, C, H, W, t0, t1)
